# initial kernel scaffold (unmeasured)
import jax
import jax.numpy as jnp
from jax import lax
from jax.experimental import pallas as pl
from jax.experimental.pallas import tpu as pltpu

N_DEV = 4


def _gather_body(x_ref, w_ref, xg_ref, wg_ref,
                 send_x, recv_x, send_w, recv_w, local_sem):
    my = lax.axis_index("i")

    barrier = pltpu.get_barrier_semaphore()
    for k in range(1, N_DEV):
        pl.semaphore_signal(
            barrier, inc=1,
            device_id=((my + k) % N_DEV,),
            device_id_type=pl.DeviceIdType.MESH,
        )
    pl.semaphore_wait(barrier, N_DEV - 1)

    cp_x = pltpu.make_async_copy(x_ref, xg_ref.at[my], local_sem.at[0])
    cp_w = pltpu.make_async_copy(w_ref, wg_ref.at[my], local_sem.at[1])
    cp_x.start()
    cp_w.start()

    sends = []
    for k in range(1, N_DEV):
        peer = (my + k) % N_DEV
        for src, dst, ssem, rsem in (
            (x_ref, xg_ref, send_x, recv_x),
            (w_ref, wg_ref, send_w, recv_w),
        ):
            rdma = pltpu.make_async_remote_copy(
                src_ref=src,
                dst_ref=dst.at[my],
                send_sem=ssem.at[k],
                recv_sem=rsem.at[k],
                device_id=(peer,),
                device_id_type=pl.DeviceIdType.MESH,
            )
            rdma.start()
            sends.append(rdma)

    for k in range(1, N_DEV):
        src_pos = (my - k) % N_DEV
        for src, dst, ssem, rsem in (
            (x_ref, xg_ref, send_x, recv_x),
            (w_ref, wg_ref, send_w, recv_w),
        ):
            recv = pltpu.make_async_remote_copy(
                src_ref=src,
                dst_ref=dst.at[src_pos],
                send_sem=ssem.at[0],
                recv_sem=rsem.at[k],
                device_id=(my,),
                device_id_type=pl.DeviceIdType.MESH,
            )
            recv.wait_recv()

    for rdma in sends:
        rdma.wait_send()
    cp_x.wait()
    cp_w.wait()


def _gemm_body(xg_ref, wg_ref, sx_ref, sw_ref, out_ref):
    s = sx_ref[0] * sw_ref[0]
    acc = jnp.dot(xg_ref[0], wg_ref[0], preferred_element_type=jnp.float32)
    for p in range(1, N_DEV):
        acc += jnp.dot(xg_ref[p], wg_ref[p],
                       preferred_element_type=jnp.float32)
    out_ref[:, :] = jnp.maximum(acc * s, 0.0)


def kernel(x, w_mat, scale_x, scale_w):
    m, kp = x.shape
    kp2, n = w_mat.shape
    assert kp == kp2

    x8 = x.astype(jnp.float8_e5m2)
    w8 = w_mat.astype(jnp.float8_e5m2)

    xg, wg = pl.pallas_call(
        _gather_body,
        out_shape=[
            jax.ShapeDtypeStruct((N_DEV, m, kp), jnp.float8_e5m2),
            jax.ShapeDtypeStruct((N_DEV, kp, n), jnp.float8_e5m2),
        ],
        in_specs=[
            pl.BlockSpec(memory_space=pltpu.VMEM),
            pl.BlockSpec(memory_space=pltpu.VMEM),
        ],
        out_specs=[
            pl.BlockSpec(memory_space=pltpu.ANY),
            pl.BlockSpec(memory_space=pltpu.ANY),
        ],
        scratch_shapes=[
            pltpu.SemaphoreType.DMA((N_DEV,)),
            pltpu.SemaphoreType.DMA((N_DEV,)),
            pltpu.SemaphoreType.DMA((N_DEV,)),
            pltpu.SemaphoreType.DMA((N_DEV,)),
            pltpu.SemaphoreType.DMA((2,)),
        ],
        compiler_params=pltpu.CompilerParams(collective_id=0),
    )(x8, w8)

    bm, bn = 1024, 2048
    out = pl.pallas_call(
        _gemm_body,
        grid=(m // bm, n // bn),
        in_specs=[
            pl.BlockSpec((N_DEV, bm, kp), lambda i, j: (0, i, 0)),
            pl.BlockSpec((N_DEV, kp, bn), lambda i, j: (0, 0, j)),
            pl.BlockSpec(memory_space=pltpu.SMEM),
            pl.BlockSpec(memory_space=pltpu.SMEM),
        ],
        out_specs=pl.BlockSpec((bm, bn), lambda i, j: (i, j)),
        out_shape=jax.ShapeDtypeStruct((m, n), jnp.float32),
    )(xg, wg, scale_x, scale_w)
    return out


# baseline (device time: 460486 ns/iter reference)
import jax
import jax.numpy as jnp
from jax import lax
from jax.experimental import pallas as pl
from jax.experimental.pallas import tpu as pltpu

N_DEV = 4


def _gather_body(x_ref, w_ref, xg_ref, wg_ref,
                 send_x, recv_x, send_w, recv_w, local_sem):
    my = lax.axis_index("i")

    barrier = pltpu.get_barrier_semaphore()
    for k in range(1, N_DEV):
        pl.semaphore_signal(
            barrier, inc=1,
            device_id=((my + k) % N_DEV,),
            device_id_type=pl.DeviceIdType.MESH,
        )
    pl.semaphore_wait(barrier, N_DEV - 1)

    cp_x = pltpu.make_async_copy(x_ref, xg_ref.at[my], local_sem.at[0])
    cp_w = pltpu.make_async_copy(w_ref, wg_ref.at[my], local_sem.at[1])
    cp_x.start()
    cp_w.start()

    sends = []
    for k in range(1, N_DEV):
        peer = (my + k) % N_DEV
        for src, dst, ssem, rsem in (
            (x_ref, xg_ref, send_x, recv_x),
            (w_ref, wg_ref, send_w, recv_w),
        ):
            rdma = pltpu.make_async_remote_copy(
                src_ref=src,
                dst_ref=dst.at[my],
                send_sem=ssem.at[k],
                recv_sem=rsem.at[k],
                device_id=(peer,),
                device_id_type=pl.DeviceIdType.MESH,
            )
            rdma.start()
            sends.append(rdma)

    for k in range(1, N_DEV):
        src_pos = (my - k) % N_DEV
        for src, dst, ssem, rsem in (
            (x_ref, xg_ref, send_x, recv_x),
            (w_ref, wg_ref, send_w, recv_w),
        ):
            recv = pltpu.make_async_remote_copy(
                src_ref=src,
                dst_ref=dst.at[src_pos],
                send_sem=ssem.at[0],
                recv_sem=rsem.at[k],
                device_id=(my,),
                device_id_type=pl.DeviceIdType.MESH,
            )
            recv.wait_recv()

    for rdma in sends:
        rdma.wait_send()
    cp_x.wait()
    cp_w.wait()


def _gemm_body(xg_ref, wg_ref, sx_ref, sw_ref, out_ref):
    s = sx_ref[0] * sw_ref[0]
    acc = jnp.dot(xg_ref[0], wg_ref[0], preferred_element_type=jnp.float32)
    for p in range(1, N_DEV):
        acc += jnp.dot(xg_ref[p], wg_ref[p],
                       preferred_element_type=jnp.float32)
    out_ref[:, :] = jnp.maximum(acc * s, 0.0)


def kernel(x, w_mat, scale_x, scale_w):
    m, kp = x.shape
    kp2, n = w_mat.shape
    assert kp == kp2

    x8 = x.astype(jnp.float8_e5m2)
    w8 = w_mat.astype(jnp.float8_e5m2)

    xg, wg = pl.pallas_call(
        _gather_body,
        out_shape=[
            jax.ShapeDtypeStruct((N_DEV, m, kp), jnp.float8_e5m2),
            jax.ShapeDtypeStruct((N_DEV, kp, n), jnp.float8_e5m2),
        ],
        in_specs=[
            pl.BlockSpec(memory_space=pltpu.VMEM),
            pl.BlockSpec(memory_space=pltpu.VMEM),
        ],
        out_specs=[
            pl.BlockSpec(memory_space=pl.ANY),
            pl.BlockSpec(memory_space=pl.ANY),
        ],
        scratch_shapes=[
            pltpu.SemaphoreType.DMA((N_DEV,)),
            pltpu.SemaphoreType.DMA((N_DEV,)),
            pltpu.SemaphoreType.DMA((N_DEV,)),
            pltpu.SemaphoreType.DMA((N_DEV,)),
            pltpu.SemaphoreType.DMA((2,)),
        ],
        compiler_params=pltpu.CompilerParams(collective_id=0),
    )(x8, w8)

    bm, bn = 1024, 1024
    out = pl.pallas_call(
        _gemm_body,
        grid=(m // bm, n // bn),
        in_specs=[
            pl.BlockSpec((N_DEV, bm, kp), lambda i, j: (0, i, 0)),
            pl.BlockSpec((N_DEV, kp, bn), lambda i, j: (0, 0, j)),
            pl.BlockSpec(memory_space=pltpu.SMEM),
            pl.BlockSpec(memory_space=pltpu.SMEM),
        ],
        out_specs=pl.BlockSpec((bm, bn), lambda i, j: (i, j)),
        out_shape=jax.ShapeDtypeStruct((m, n), jnp.float32),
    )(xg, wg, scale_x, scale_w)
    return out
